# baseline (speedup 1.0000x reference)
import functools

import jax
import jax.numpy as jnp
from jax import lax
from jax.experimental import pallas as pl
from jax.experimental.pallas import tpu as pltpu
from jax.experimental.pallas import tpu_sc as plsc

N_NODES = 50000
D = 256
G = 256

NC = 2
NS = 16
L = 16

DC = D // NC
CHUNK = 80
GROUP = 320
CPG = GROUP // CHUNK
ROWS_PER_TILE = 3200
NGROUP = ROWS_PER_TILE // GROUP
NCHUNK = N_NODES // CHUNK

PAD_N = 50176
CBW = 512


@functools.partial(
    pl.kernel,
    out_type=jax.ShapeDtypeStruct((G, D), jnp.float32),
    mesh=plsc.VectorSubcoreMesh(core_axis_name="c", subcore_axis_name="s"),
    scratch_types=[
        pltpu.VMEM((ROWS_PER_TILE // CHUNK, CHUNK), jnp.int32),
        pltpu.VMEM((GROUP, DC), jnp.float32),
        pltpu.VMEM((GROUP, DC), jnp.float32),
        pltpu.VMEM((L, DC), jnp.float32),
        pltpu.SemaphoreType.DMA,
        pltpu.SemaphoreType.DMA,
        pltpu.VMEM_SHARED((G, DC), jnp.float32),
    ],
)
def _segment_sums_sc(x_hbm, seg_hbm, out_hbm, segb, xb0, xb1, zb,
                     sem0, sem1, acc_sh):
    c = lax.axis_index("c")
    s = lax.axis_index("s")
    row_base = s * ROWS_PER_TILE
    col0 = c * DC
    chunk0 = s * (ROWS_PER_TILE // CHUNK)

    zero_v = jnp.zeros((L,), dtype=jnp.float32)
    for r in range(L):
        for j in range(DC // L):
            zb[r, pl.ds(j * L, L)] = zero_v
    pltpu.sync_copy(zb, acc_sh.at[pl.ds(s * L, L), :])

    pltpu.sync_copy(seg_hbm.at[pl.ds(chunk0, ROWS_PER_TILE // CHUNK), :], segb)

    bufs = (xb0, xb1)
    sems = (sem0, sem1)

    def grp_rows(g):
        return row_base + g * GROUP

    def full(g):
        return grp_rows(g) + GROUP <= N_NODES

    def start_load(g):
        buf = bufs[g % 2]
        sem = sems[g % 2]
        pltpu.async_copy(x_hbm.at[pl.ds(grp_rows(g), GROUP), pl.ds(col0, DC)],
                         buf, sem)

    def wait_load(g):
        buf = bufs[g % 2]
        sem = sems[g % 2]
        pltpu.make_async_copy(
            x_hbm.at[pl.ds(grp_rows(g), GROUP), pl.ds(col0, DC)],
            buf, sem).wait()

    @pl.when(full(0))
    def _():
        start_load(0)

    plsc.subcore_barrier()

    for g in range(NGROUP):
        buf = bufs[g % 2]
        if g + 1 < NGROUP:
            @pl.when(full(g + 1))
            def _(g=g):
                start_load(g + 1)

        @pl.when(full(g))
        def _(g=g, buf=buf):
            wait_load(g)
            for q in range(CPG):
                pltpu.sync_copy(buf.at[pl.ds(q * CHUNK, CHUNK), :],
                                acc_sh.at[segb.at[g * CPG + q]], add=True)

        @pl.when(jnp.logical_and(jnp.logical_not(full(g)),
                                 grp_rows(g) + CHUNK <= N_NODES))
        def _(g=g, buf=buf):
            for q in range(CPG):
                @pl.when(grp_rows(g) + (q + 1) * CHUNK <= N_NODES)
                def _(g=g, q=q, buf=buf):
                    pltpu.sync_copy(
                        x_hbm.at[pl.ds(grp_rows(g) + q * CHUNK, CHUNK),
                                 pl.ds(col0, DC)],
                        buf.at[pl.ds(q * CHUNK, CHUNK), :])
                    pltpu.sync_copy(buf.at[pl.ds(q * CHUNK, CHUNK), :],
                                    acc_sh.at[segb.at[g * CPG + q]], add=True)

    plsc.subcore_barrier()
    pltpu.sync_copy(acc_sh.at[pl.ds(s * L, L), :],
                    out_hbm.at[pl.ds(s * L, L), pl.ds(col0, DC)])


def _mean_body(seg_ref, sums_ref, out_ref, acc_ref):
    i = pl.program_id(0)

    @pl.when(i == 0)
    def _():
        acc_ref[...] = jnp.zeros_like(acc_ref)

    ids = seg_ref[...]
    gcol = lax.broadcasted_iota(jnp.int32, (G, 1), 0)
    acc_ref[...] += (ids == gcol).astype(jnp.float32)

    @pl.when(i == pl.num_programs(0) - 1)
    def _():
        cnt = jnp.sum(acc_ref[...], axis=1, keepdims=True)
        inv = 1.0 / jnp.maximum(cnt, 1.0)
        out_ref[...] = sums_ref[...] * inv


def _tc_mean(seg_row, sums):
    return pl.pallas_call(
        _mean_body,
        grid=(PAD_N // CBW,),
        in_specs=[pl.BlockSpec((1, CBW), lambda i: (0, i)),
                  pl.BlockSpec((G, D), lambda i: (0, 0))],
        out_specs=pl.BlockSpec((G, D), lambda i: (0, 0)),
        out_shape=jax.ShapeDtypeStruct((G, D), jnp.float32),
        scratch_shapes=[pltpu.VMEM((G, CBW), jnp.float32)],
    )(seg_row, sums)


def kernel(x, segment_ids):
    seg = segment_ids.astype(jnp.int32)
    seg2d = jnp.pad(seg.reshape(NCHUNK, CHUNK),
                    ((0, NS * (ROWS_PER_TILE // CHUNK) - NCHUNK), (0, 0)))
    seg_row = jnp.pad(seg, (0, PAD_N - N_NODES),
                      constant_values=G).reshape(1, PAD_N)
    sums = _segment_sums_sc(x, seg2d)
    return _tc_mean(seg_row, sums)


# speedup vs baseline: 1.6973x; 1.6973x over previous
import functools

import jax
import jax.numpy as jnp
from jax import lax
from jax.experimental import pallas as pl
from jax.experimental.pallas import tpu as pltpu
from jax.experimental.pallas import tpu_sc as plsc

N_NODES = 50000
D = 256
G = 256

NC = 2
NS = 16
L = 16

DC = D // NC
CHUNK = 80
GROUP = 320
CPG = GROUP // CHUNK
ROWS_PER_TILE = 3200
NGROUP = ROWS_PER_TILE // GROUP
NCHUNK = N_NODES // CHUNK

PAD_N = 51200
CBW = 1024


@functools.partial(
    pl.kernel,
    out_type=jax.ShapeDtypeStruct((G, D), jnp.float32),
    mesh=plsc.VectorSubcoreMesh(core_axis_name="c", subcore_axis_name="s"),
    scratch_types=[
        pltpu.VMEM((ROWS_PER_TILE // CHUNK, CHUNK), jnp.int32),
        pltpu.VMEM((GROUP, DC), jnp.float32),
        pltpu.VMEM((GROUP, DC), jnp.float32),
        pltpu.VMEM((L, DC), jnp.float32),
        pltpu.SemaphoreType.DMA,
        pltpu.SemaphoreType.DMA,
        pltpu.SemaphoreType.DMA,
        pltpu.VMEM_SHARED((G, DC), jnp.float32),
    ],
)
def _segment_sums_sc(x_hbm, seg_hbm, out_hbm, segb, xb0, xb1, zb,
                     sem0, sem1, sem2, acc_sh):
    c = lax.axis_index("c")
    s = lax.axis_index("s")
    row_base = s * ROWS_PER_TILE
    col0 = c * DC
    chunk0 = s * (ROWS_PER_TILE // CHUNK)

    zero_v = jnp.zeros((L,), dtype=jnp.float32)
    for r in range(L):
        for j in range(DC // L):
            zb[r, pl.ds(j * L, L)] = zero_v
    pltpu.sync_copy(zb, acc_sh.at[pl.ds(s * L, L), :])

    pltpu.sync_copy(seg_hbm.at[pl.ds(chunk0, ROWS_PER_TILE // CHUNK), :], segb)

    bufs = (xb0, xb1)
    sems = (sem0, sem1)

    def grp_rows(g):
        return row_base + g * GROUP

    def full(g):
        return grp_rows(g) + GROUP <= N_NODES

    def start_load(g):
        buf = bufs[g % 2]
        sem = sems[g % 2]
        pltpu.async_copy(x_hbm.at[pl.ds(grp_rows(g), GROUP), pl.ds(col0, DC)],
                         buf, sem)

    def wait_load(g):
        buf = bufs[g % 2]
        sem = sems[g % 2]
        pltpu.make_async_copy(
            x_hbm.at[pl.ds(grp_rows(g), GROUP), pl.ds(col0, DC)],
            buf, sem).wait()

    @pl.when(full(0))
    def _():
        start_load(0)

    plsc.subcore_barrier()

    for g in range(NGROUP):
        buf = bufs[g % 2]
        if g + 1 < NGROUP:
            @pl.when(full(g + 1))
            def _(g=g):
                start_load(g + 1)

        @pl.when(full(g))
        def _(g=g, buf=buf):
            wait_load(g)
            handles = []
            for q in range(CPG):
                handles.append(pltpu.async_copy(
                    buf.at[pl.ds(q * CHUNK, CHUNK), :],
                    acc_sh.at[segb.at[g * CPG + q]], sem2, add=True))
            for h in handles:
                h.wait()

        @pl.when(jnp.logical_and(jnp.logical_not(full(g)),
                                 grp_rows(g) + CHUNK <= N_NODES))
        def _(g=g, buf=buf):
            for q in range(CPG):
                @pl.when(grp_rows(g) + (q + 1) * CHUNK <= N_NODES)
                def _(g=g, q=q, buf=buf):
                    pltpu.sync_copy(
                        x_hbm.at[pl.ds(grp_rows(g) + q * CHUNK, CHUNK),
                                 pl.ds(col0, DC)],
                        buf.at[pl.ds(q * CHUNK, CHUNK), :])
                    pltpu.sync_copy(buf.at[pl.ds(q * CHUNK, CHUNK), :],
                                    acc_sh.at[segb.at[g * CPG + q]], add=True)

    plsc.subcore_barrier()
    pltpu.sync_copy(acc_sh.at[pl.ds(s * L, L), :],
                    out_hbm.at[pl.ds(s * L, L), pl.ds(col0, DC)])


def _counts_body(seg_ref, cnt_ref, acc_ref):
    i = pl.program_id(0)

    @pl.when(i == 0)
    def _():
        acc_ref[...] = jnp.zeros_like(acc_ref)

    ids = seg_ref[...]
    grow = lax.broadcasted_iota(jnp.int32, (16, 1), 0)
    a = ((ids >> 4) == grow).astype(jnp.float32)
    b = ((ids & 15) == grow).astype(jnp.float32)
    acc_ref[...] += lax.dot_general(
        a, b, (((1,), (1,)), ((), ())),
        preferred_element_type=jnp.float32)

    @pl.when(i == pl.num_programs(0) - 1)
    def _():
        cnt_ref[...] = acc_ref[...]


def _tc_counts(seg_row):
    return pl.pallas_call(
        _counts_body,
        grid=(PAD_N // CBW,),
        in_specs=[pl.BlockSpec((1, CBW), lambda i: (0, i))],
        out_specs=pl.BlockSpec((16, 16), lambda i: (0, 0)),
        out_shape=jax.ShapeDtypeStruct((16, 16), jnp.float32),
        scratch_shapes=[pltpu.VMEM((16, 16), jnp.float32)],
    )(seg_row)


def _scale_body(sums_ref, cnt_ref, out_ref):
    inv = 1.0 / jnp.maximum(cnt_ref[...], 1.0)
    out_ref[...] = sums_ref[...] * inv


def _tc_scale(sums, cnt_col):
    return pl.pallas_call(
        _scale_body,
        out_shape=jax.ShapeDtypeStruct((G, D), jnp.float32),
    )(sums, cnt_col)


def kernel(x, segment_ids):
    seg = segment_ids.astype(jnp.int32)
    seg2d = jnp.pad(seg.reshape(NCHUNK, CHUNK),
                    ((0, NS * (ROWS_PER_TILE // CHUNK) - NCHUNK), (0, 0)))
    seg_row = jnp.pad(seg, (0, PAD_N - N_NODES),
                      constant_values=G).reshape(1, PAD_N)
    sums = _segment_sums_sc(x, seg2d)
    cnt_col = _tc_counts(seg_row).reshape(G, 1)
    return _tc_scale(sums, cnt_col)
